# baseline (speedup 1.0000x reference)
import functools

import jax
import jax.numpy as jnp
from jax import lax
from jax.experimental import pallas as pl
from jax.experimental.pallas import tpu as pltpu
from jax.experimental.pallas import tpu_sc as plsc

H = 512
B = 16384

NC, NS, L = 2, 16, 16
NW = NC * NS
RPW = B // NW
C = 32
NCHUNK = RPW // C
HC = H // L


def _neg_log_sigmoid(x):
    t = jnp.exp(-jnp.abs(x))
    w = t / (2.0 + t)
    w2 = w * w
    p = 2.0 * w * (1.0 + w2 * (1.0 / 3.0 + w2 * (0.2 + w2 * (1.0 / 7.0))))
    return p - jnp.minimum(x, 0.0)


def _sc_partials(user_table, item_table, user, item, neg):
    mesh = plsc.VectorSubcoreMesh(core_axis_name="c", subcore_axis_name="s")

    @functools.partial(
        pl.kernel,
        mesh=mesh,
        out_type=jax.ShapeDtypeStruct((NW * L,), jnp.float32),
        compiler_params=pltpu.CompilerParams(use_tc_tiling_on_sc=True,
                                             needs_layout_passes=False),
        scratch_types=[
            pltpu.VMEM((RPW,), jnp.int32),
            pltpu.VMEM((RPW,), jnp.int32),
            pltpu.VMEM((RPW,), jnp.int32),
            pltpu.VMEM((C, H), jnp.float32),
            pltpu.VMEM((C, H), jnp.float32),
            pltpu.VMEM((C, H), jnp.float32),
            pltpu.VMEM((C, H), jnp.float32),
            pltpu.VMEM((C, H), jnp.float32),
            pltpu.VMEM((C, H), jnp.float32),
            pltpu.VMEM((L,), jnp.float32),
            pltpu.SemaphoreType.DMA,
            pltpu.SemaphoreType.DMA,
        ],
    )
    def k(ut_hbm, it_hbm, u_hbm, i_hbm, n_hbm, out_hbm,
          uidx, iidx, nidx, ub0, ib0, nb0, ub1, ib1, nb1, lossv, s0, s1):
        wid = lax.axis_index("s") * NC + lax.axis_index("c")
        base = wid * RPW
        pltpu.sync_copy(u_hbm.at[pl.ds(base, RPW)], uidx)
        pltpu.sync_copy(i_hbm.at[pl.ds(base, RPW)], iidx)
        pltpu.sync_copy(n_hbm.at[pl.ds(base, RPW)], nidx)

        bufs = ((ub0, ib0, nb0, s0), (ub1, ib1, nb1, s1))
        lane = lax.iota(jnp.int32, L)

        def issue(c, par):
            ub, ib, nb, sem = bufs[par]
            off = pl.multiple_of(c * C, C)
            pltpu.async_copy(ut_hbm.at[uidx.at[pl.ds(off, C)]], ub, sem)
            pltpu.async_copy(it_hbm.at[iidx.at[pl.ds(off, C)]], ib, sem)
            pltpu.async_copy(it_hbm.at[nidx.at[pl.ds(off, C)]], nb, sem)

        def drain(par):
            ub, ib, nb, sem = bufs[par]
            pltpu.make_async_copy(ut_hbm.at[pl.ds(0, C)], ub, sem).wait()
            pltpu.make_async_copy(it_hbm.at[pl.ds(0, C)], ib, sem).wait()
            pltpu.make_async_copy(it_hbm.at[pl.ds(0, C)], nb, sem).wait()

        issue(0, 0)

        def chunk_body(cc, loss):
            for par in range(2):
                c = cc + par
                ub, ib, nb, _sem = bufs[par]

                @pl.when(c + 1 < NCHUNK)
                def _issue_next():
                    issue(c + 1, 1 - par)

                drain(par)

                for g in range(C // L):
                    def step(t, accs, g=g):
                        hs = pl.multiple_of(t * L, L)
                        out = []
                        for r in range(L):
                            row = g * L + r
                            uu = ub[row, pl.ds(hs, L)]
                            ii = ib[row, pl.ds(hs, L)]
                            nn = nb[row, pl.ds(hs, L)]
                            out.append(accs[r] + uu * (ii - nn))
                        return tuple(out)

                    accs = lax.fori_loop(
                        0, HC, step,
                        tuple(jnp.zeros((L,), jnp.float32) for _ in range(L)))
                    res = jnp.zeros((L,), jnp.float32)
                    for r in range(L):
                        res = jnp.where(lane == r, jnp.sum(accs[r]), res)
                    loss = loss + _neg_log_sigmoid(res)
            return loss

        loss = lax.fori_loop(0, NCHUNK // 2,
                             lambda i, x: chunk_body(i * 2, x),
                             jnp.zeros((L,), jnp.float32))
        lossv[...] = loss
        pltpu.sync_copy(lossv, out_hbm.at[pl.ds(wid * L, L)])

    return k(user_table, item_table, user, item, neg)


def _tc_sum(partials):
    def body(p_ref, o_ref):
        o_ref[0, 0] = jnp.sum(p_ref[...])

    out = pl.pallas_call(
        body,
        out_shape=jax.ShapeDtypeStruct((1, 1), jnp.float32),
        out_specs=pl.BlockSpec(memory_space=pltpu.SMEM),
    )(partials)
    return out[0, 0]


def kernel(user_table, item_table, user, item, neg):
    partials = _sc_partials(user_table, item_table,
                            user.astype(jnp.int32), item.astype(jnp.int32),
                            neg.astype(jnp.int32))
    return _tc_sum(partials)


# speedup vs baseline: 1.0022x; 1.0022x over previous
import functools

import jax
import jax.numpy as jnp
from jax import lax
from jax.experimental import pallas as pl
from jax.experimental.pallas import tpu as pltpu
from jax.experimental.pallas import tpu_sc as plsc

H = 512
B = 16384

NC, NS, L = 2, 16, 16
NW = NC * NS
RPW = B // NW
C = 32
NCHUNK = RPW // C
HC = H // L


def _neg_log_sigmoid(x):
    t = jnp.exp(-jnp.abs(x))
    w = t / (2.0 + t)
    w2 = w * w
    p = 2.0 * w * (1.0 + w2 * (1.0 / 3.0 + w2 * (0.2 + w2 * (1.0 / 7.0))))
    return p - jnp.minimum(x, 0.0)


def _sc_partials(user_table, item_table, user, item, neg):
    mesh = plsc.VectorSubcoreMesh(core_axis_name="c", subcore_axis_name="s")

    @functools.partial(
        pl.kernel,
        mesh=mesh,
        out_type=jax.ShapeDtypeStruct((NW * L,), jnp.float32),
        compiler_params=pltpu.CompilerParams(use_tc_tiling_on_sc=True,
                                             needs_layout_passes=False,
                                             disable_bounds_checks=True,
                                             disable_semaphore_checks=True,
                                             skip_device_barrier=True),
        scratch_types=[
            pltpu.VMEM((RPW,), jnp.int32),
            pltpu.VMEM((RPW,), jnp.int32),
            pltpu.VMEM((RPW,), jnp.int32),
            pltpu.VMEM((C, H), jnp.float32),
            pltpu.VMEM((C, H), jnp.float32),
            pltpu.VMEM((C, H), jnp.float32),
            pltpu.VMEM((C, H), jnp.float32),
            pltpu.VMEM((C, H), jnp.float32),
            pltpu.VMEM((C, H), jnp.float32),
            pltpu.VMEM((L,), jnp.float32),
            pltpu.SemaphoreType.DMA,
            pltpu.SemaphoreType.DMA,
        ],
    )
    def k(ut_hbm, it_hbm, u_hbm, i_hbm, n_hbm, out_hbm,
          uidx, iidx, nidx, ub0, ib0, nb0, ub1, ib1, nb1, lossv, s0, s1):
        wid = lax.axis_index("s") * NC + lax.axis_index("c")
        base = wid * RPW
        pltpu.sync_copy(u_hbm.at[pl.ds(base, RPW)], uidx)
        pltpu.sync_copy(i_hbm.at[pl.ds(base, RPW)], iidx)
        pltpu.sync_copy(n_hbm.at[pl.ds(base, RPW)], nidx)

        bufs = ((ub0, ib0, nb0, s0), (ub1, ib1, nb1, s1))
        lane = lax.iota(jnp.int32, L)

        def issue(c, par):
            ub, ib, nb, sem = bufs[par]
            off = pl.multiple_of(c * C, C)
            pltpu.async_copy(ut_hbm.at[uidx.at[pl.ds(off, C)]], ub, sem)
            pltpu.async_copy(it_hbm.at[iidx.at[pl.ds(off, C)]], ib, sem)
            pltpu.async_copy(it_hbm.at[nidx.at[pl.ds(off, C)]], nb, sem)

        def drain(par):
            ub, ib, nb, sem = bufs[par]
            pltpu.make_async_copy(ut_hbm.at[pl.ds(0, C)], ub, sem).wait()
            pltpu.make_async_copy(it_hbm.at[pl.ds(0, C)], ib, sem).wait()
            pltpu.make_async_copy(it_hbm.at[pl.ds(0, C)], nb, sem).wait()

        issue(0, 0)

        def chunk_body(cc, loss):
            for par in range(2):
                c = cc + par
                ub, ib, nb, _sem = bufs[par]

                @pl.when(c + 1 < NCHUNK)
                def _issue_next():
                    issue(c + 1, 1 - par)

                drain(par)

                for g in range(C // L):
                    def step(t, accs, g=g):
                        hs = pl.multiple_of(t * L, L)
                        out = []
                        for r in range(L):
                            row = g * L + r
                            uu = ub[row, pl.ds(hs, L)]
                            ii = ib[row, pl.ds(hs, L)]
                            nn = nb[row, pl.ds(hs, L)]
                            out.append(accs[r] + uu * (ii - nn))
                        return tuple(out)

                    accs = lax.fori_loop(
                        0, HC, step,
                        tuple(jnp.zeros((L,), jnp.float32) for _ in range(L)))
                    res = jnp.zeros((L,), jnp.float32)
                    for r in range(L):
                        res = jnp.where(lane == r, jnp.sum(accs[r]), res)
                    loss = loss + _neg_log_sigmoid(res)
            return loss

        loss = lax.fori_loop(0, NCHUNK // 2,
                             lambda i, x: chunk_body(i * 2, x),
                             jnp.zeros((L,), jnp.float32))
        lossv[...] = loss
        pltpu.sync_copy(lossv, out_hbm.at[pl.ds(wid * L, L)])

    return k(user_table, item_table, user, item, neg)


def _tc_sum(partials):
    def body(p_ref, o_ref):
        o_ref[0, 0] = jnp.sum(p_ref[...])

    out = pl.pallas_call(
        body,
        out_shape=jax.ShapeDtypeStruct((1, 1), jnp.float32),
        out_specs=pl.BlockSpec(memory_space=pltpu.SMEM),
    )(partials)
    return out[0, 0]


def kernel(user_table, item_table, user, item, neg):
    partials = _sc_partials(user_table, item_table,
                            user.astype(jnp.int32), item.astype(jnp.int32),
                            neg.astype(jnp.int32))
    return _tc_sum(partials)


# speedup vs baseline: 1.0731x; 1.0708x over previous
import functools

import jax
import jax.numpy as jnp
from jax import lax
from jax.experimental import pallas as pl
from jax.experimental.pallas import tpu as pltpu
from jax.experimental.pallas import tpu_sc as plsc

H = 512
B = 16384

NC, NS, L = 2, 16, 16
NW = NC * NS
RPW = B // NW
C = 16
NCHUNK = RPW // C
DEPTH = 4
HC = H // L


def _neg_log_sigmoid(x):
    t = jnp.exp(-jnp.abs(x))
    w = t / (2.0 + t)
    w2 = w * w
    p = 2.0 * w * (1.0 + w2 * (1.0 / 3.0 + w2 * (0.2 + w2 * (1.0 / 7.0))))
    return p - jnp.minimum(x, 0.0)


def _sc_partials(user_table, item_table, user, item, neg):
    mesh = plsc.VectorSubcoreMesh(core_axis_name="c", subcore_axis_name="s")

    @functools.partial(
        pl.kernel,
        mesh=mesh,
        out_type=jax.ShapeDtypeStruct((NW * L,), jnp.float32),
        compiler_params=pltpu.CompilerParams(use_tc_tiling_on_sc=True,
                                             needs_layout_passes=False,
                                             disable_bounds_checks=True,
                                             disable_semaphore_checks=True,
                                             skip_device_barrier=True),
        scratch_types=(
            [pltpu.VMEM((RPW,), jnp.int32) for _ in range(3)]
            + [pltpu.VMEM((C, H), jnp.float32) for _ in range(3 * DEPTH)]
            + [pltpu.VMEM((L,), jnp.float32)]
            + [pltpu.SemaphoreType.DMA for _ in range(DEPTH)]
        ),
    )
    def k(ut_hbm, it_hbm, u_hbm, i_hbm, n_hbm, out_hbm, *refs):
        uidx, iidx, nidx = refs[0:3]
        rowbufs = refs[3:3 + 3 * DEPTH]
        lossv = refs[3 + 3 * DEPTH]
        sems = refs[4 + 3 * DEPTH:]
        bufs = tuple((rowbufs[3 * p], rowbufs[3 * p + 1], rowbufs[3 * p + 2],
                      sems[p]) for p in range(DEPTH))

        wid = lax.axis_index("s") * NC + lax.axis_index("c")
        base = wid * RPW
        pltpu.sync_copy(u_hbm.at[pl.ds(base, RPW)], uidx)
        pltpu.sync_copy(i_hbm.at[pl.ds(base, RPW)], iidx)
        pltpu.sync_copy(n_hbm.at[pl.ds(base, RPW)], nidx)

        lane = lax.iota(jnp.int32, L)

        def issue(c, par):
            ub, ib, nb, sem = bufs[par]
            off = pl.multiple_of(c * C, C)
            pltpu.async_copy(ut_hbm.at[uidx.at[pl.ds(off, C)]], ub, sem)
            pltpu.async_copy(it_hbm.at[iidx.at[pl.ds(off, C)]], ib, sem)
            pltpu.async_copy(it_hbm.at[nidx.at[pl.ds(off, C)]], nb, sem)

        def drain(par):
            ub, ib, nb, sem = bufs[par]
            pltpu.make_async_copy(ut_hbm.at[pl.ds(0, C)], ub, sem).wait()
            pltpu.make_async_copy(it_hbm.at[pl.ds(0, C)], ib, sem).wait()
            pltpu.make_async_copy(it_hbm.at[pl.ds(0, C)], nb, sem).wait()

        for p in range(DEPTH - 1):
            issue(p, p)

        def chunk_body(cc, loss):
            for par in range(DEPTH):
                c = cc + par
                ub, ib, nb, _sem = bufs[par]

                @pl.when(c + DEPTH - 1 < NCHUNK)
                def _issue_next():
                    issue(c + DEPTH - 1, (par + DEPTH - 1) % DEPTH)

                drain(par)

                def step(t, accs):
                    hs = pl.multiple_of(t * L, L)
                    out = []
                    for r in range(L):
                        uu = ub[r, pl.ds(hs, L)]
                        ii = ib[r, pl.ds(hs, L)]
                        nn = nb[r, pl.ds(hs, L)]
                        out.append(accs[r] + uu * (ii - nn))
                    return tuple(out)

                accs = lax.fori_loop(
                    0, HC, step,
                    tuple(jnp.zeros((L,), jnp.float32) for _ in range(L)))
                res = jnp.zeros((L,), jnp.float32)
                for r in range(L):
                    res = jnp.where(lane == r, jnp.sum(accs[r]), res)
                loss = loss + _neg_log_sigmoid(res)
            return loss

        loss = lax.fori_loop(0, NCHUNK // DEPTH,
                             lambda i, x: chunk_body(i * DEPTH, x),
                             jnp.zeros((L,), jnp.float32))
        lossv[...] = loss
        pltpu.sync_copy(lossv, out_hbm.at[pl.ds(wid * L, L)])

    return k(user_table, item_table, user, item, neg)


def _tc_sum(partials):
    def body(p_ref, o_ref):
        o_ref[0, 0] = jnp.sum(p_ref[...])

    out = pl.pallas_call(
        body,
        out_shape=jax.ShapeDtypeStruct((1, 1), jnp.float32),
        out_specs=pl.BlockSpec(memory_space=pltpu.SMEM),
    )(partials)
    return out[0, 0]


def kernel(user_table, item_table, user, item, neg):
    partials = _sc_partials(user_table, item_table,
                            user.astype(jnp.int32), item.astype(jnp.int32),
                            neg.astype(jnp.int32))
    return _tc_sum(partials)
